# baseline (speedup 1.0000x reference)
import functools

import jax
import jax.numpy as jnp
from jax import lax
from jax.experimental import pallas as pl
from jax.experimental.pallas import tpu as pltpu
from jax.experimental.pallas import tpu_sc as plsc

_NC = 2
_NS = 16
_CHUNK = 512


def _sc_gather(table_hbm, idx_hbm, out_hbm, idx_v, rows_v, sem, *, per_w):
    wid = lax.axis_index("s") * _NC + lax.axis_index("c")
    base = wid * per_w
    for ci in range(per_w // _CHUNK):
        off = base + ci * _CHUNK
        pltpu.sync_copy(idx_hbm.at[pl.ds(off, _CHUNK)], idx_v)
        pltpu.async_copy(table_hbm.at[idx_v], rows_v, sem).wait()
        pltpu.sync_copy(rows_v, out_hbm.at[pl.ds(off, _CHUNK)])


def _base_lookup(logP0, x, T, N):
    lp_pad = jnp.pad(logP0, ((0, 0), (0, 128 - N)))
    per_w = T // (_NC * _NS)
    mesh = plsc.VectorSubcoreMesh(core_axis_name="c", subcore_axis_name="s")
    fn = functools.partial(
        pl.kernel,
        mesh=mesh,
        out_type=jax.ShapeDtypeStruct((T, 128), jnp.float32),
        scratch_types=[
            pltpu.VMEM((_CHUNK,), jnp.int32),
            pltpu.VMEM((_CHUNK, 128), jnp.float32),
            pltpu.SemaphoreType.DMA,
        ],
    )(functools.partial(_sc_gather, per_w=per_w))
    return fn(lp_pad, x)


def _tc_body(x_ref, ut_ref, base_ref, wm_ref, o_ref, *, TB, N, U, CH):
    for i in range(TB // CH):
        sl = pl.ds(i * CH, CH)
        x = x_ref[0, 0, sl]
        s_iota = jax.lax.broadcasted_iota(jnp.int32, (N, CH), 0)
        eq = s_iota == x[None, :]
        onehot_bf = jnp.where(eq, jnp.float32(1.0),
                              jnp.float32(0.0)).astype(jnp.bfloat16)
        ut_bf = ut_ref[:, sl]
        parts = [onehot_bf * ut_bf[d:d + 1, :] for d in range(U)]
        zt = jnp.concatenate(parts, axis=0)
        dn = (((0,), (0,)), ((), ()))
        stim = jax.lax.dot_general(zt, wm_ref[...], dn,
                                   preferred_element_type=jnp.float32)
        logits = stim + base_ref[sl, :][:, :N]
        m = jnp.max(logits, axis=1, keepdims=True)
        ex = jnp.exp(logits - m)
        lz = jnp.log(jnp.sum(ex, axis=1, keepdims=True)) + m
        o_ref[sl, :] = logits - lz


@functools.partial(jax.jit, static_argnames=("tb", "ch"))
def kernel(x_curr, u_curr, logP0, W, tb=4096, ch=2048):
    T = x_curr.shape[0]
    N = logP0.shape[0]
    U = u_curr.shape[1]
    x_i32 = x_curr.astype(jnp.int32)
    base = _base_lookup(logP0, x_i32, T, N)

    cols = jnp.arange(N)[None, :]
    srows = jnp.arange(N)[:, None]
    k = jnp.clip(cols - (cols > srows).astype(jnp.int32), 0, N - 2)
    W64 = jnp.take_along_axis(W, k[:, :, None], axis=1)
    W64 = jnp.where((cols == srows)[:, :, None], 0.0, W64)
    Wm = W64.transpose(2, 0, 1).reshape(U * N, N).astype(jnp.bfloat16)

    TB = tb
    NB = T // TB
    x3 = x_i32.reshape(NB, 1, TB)
    uT = u_curr.T.astype(jnp.bfloat16)
    out = pl.pallas_call(
        functools.partial(_tc_body, TB=TB, N=N, U=U, CH=ch),
        grid=(NB,),
        in_specs=[
            pl.BlockSpec((1, 1, TB), lambda i: (i, 0, 0)),
            pl.BlockSpec((U, TB), lambda i: (0, i)),
            pl.BlockSpec((TB, 128), lambda i: (i, 0)),
            pl.BlockSpec((U * N, N), lambda i: (0, 0)),
        ],
        out_specs=pl.BlockSpec((TB, N), lambda i: (i, 0)),
        out_shape=jax.ShapeDtypeStruct((T, N), jnp.float32),
        compiler_params=pltpu.CompilerParams(
            dimension_semantics=("arbitrary",),
        ),
    )(x3, uT, base, Wm)
    return out


# speedup vs baseline: 1.8877x; 1.8877x over previous
import functools

import jax
import jax.numpy as jnp
from jax.experimental import pallas as pl
from jax.experimental.pallas import tpu as pltpu


def _body(x_ref, ut_ref, wm_ref, o_ref, *, TB, N, U, CH):
    for i in range(TB // CH):
        sl = pl.ds(i * CH, CH)
        x = x_ref[0, 0, sl]
        s_iota = jax.lax.broadcasted_iota(jnp.int32, (N, CH), 0)
        eq = s_iota == x[None, :]
        onehot_bf = jnp.where(eq, jnp.float32(1.0),
                              jnp.float32(0.0)).astype(jnp.bfloat16)
        ut_bf = ut_ref[:, sl]
        parts = [onehot_bf * ut_bf[d:d + 1, :] for d in range(U)]
        zt = jnp.concatenate(parts + [onehot_bf], axis=0)
        dn = (((0,), (0,)), ((), ()))
        logits = jax.lax.dot_general(zt, wm_ref[...], dn,
                                     preferred_element_type=jnp.float32)
        m = jnp.max(logits, axis=1, keepdims=True)
        ex = jnp.exp(logits - m)
        lz = jnp.log(jnp.sum(ex, axis=1, keepdims=True)) + m
        o_ref[sl, :] = logits - lz


@functools.partial(jax.jit, static_argnames=("interpret", "tb", "ch"))
def kernel(x_curr, u_curr, logP0, W, interpret=False, tb=4096, ch=2048):
    T = x_curr.shape[0]
    N = logP0.shape[0]
    U = u_curr.shape[1]
    cols = jnp.arange(N)[None, :]
    srows = jnp.arange(N)[:, None]
    k = jnp.clip(cols - (cols > srows).astype(jnp.int32), 0, N - 2)
    W64 = jnp.take_along_axis(W, k[:, :, None], axis=1)
    W64 = jnp.where((cols == srows)[:, :, None], 0.0, W64)
    Wm = W64.transpose(2, 0, 1).reshape(U * N, N)
    Wtot = jnp.concatenate([Wm, logP0], axis=0).astype(jnp.bfloat16)

    TB = tb
    NB = T // TB
    x3 = x_curr.astype(jnp.int32).reshape(NB, 1, TB)
    uT = u_curr.T.astype(jnp.bfloat16)
    out = pl.pallas_call(
        functools.partial(_body, TB=TB, N=N, U=U, CH=ch),
        grid=(NB,),
        in_specs=[
            pl.BlockSpec((1, 1, TB), lambda i: (i, 0, 0)),
            pl.BlockSpec((U, TB), lambda i: (0, i)),
            pl.BlockSpec((N * U + N, N), lambda i: (0, 0)),
        ],
        out_specs=pl.BlockSpec((TB, N), lambda i: (i, 0)),
        out_shape=jax.ShapeDtypeStruct((T, N), jnp.float32),
        compiler_params=pltpu.CompilerParams(
            dimension_semantics=("arbitrary",),
        ),
        interpret=interpret,
    )(x3, uT, Wtot)
    return out
